# baseline (speedup 1.0000x reference)
import functools

import jax
import jax.numpy as jnp
from jax import lax
from jax.experimental import pallas as pl
from jax.experimental.pallas import tpu as pltpu
from jax.experimental.pallas import tpu_sc as plsc

_N = 10000
_E = 320000
_D = 128
_NHEADS = 4
_ALPHA = 0.2

_NC = 2
_NS = 16
_NW = _NC * _NS
_EPW = _E // _NW
_C = 80
_NCHUNK = _EPW // _C

_BN = 2000


def _tc_proj(xt, xp, pc, pp, w):
    def body(xt_ref, xp_ref, pc_ref, pp_ref, w_ref, g_ref):
        f32 = jnp.float32
        a0 = jnp.concatenate(
            [jnp.dot(pc_ref[h], w_ref[h, 0:32, :], preferred_element_type=f32)
             for h in range(_NHEADS)], axis=1)
        a1 = jnp.concatenate(
            [jnp.dot(pc_ref[h], w_ref[h, 32:64, :], preferred_element_type=f32)
             for h in range(_NHEADS)], axis=1)
        a2 = jnp.concatenate(
            [jnp.dot(pp_ref[h], w_ref[h, 64:96, :], preferred_element_type=f32)
             for h in range(_NHEADS)], axis=1)
        xt_blk = xt_ref[...]
        g_ref[0] = jnp.dot(xt_blk, a0, preferred_element_type=f32)
        g_ref[1] = jnp.dot(xt_blk, a1, preferred_element_type=f32)
        g_ref[2] = jnp.dot(xp_ref[...], a2, preferred_element_type=f32)

    return pl.pallas_call(
        body,
        grid=(_N // _BN,),
        in_specs=[
            pl.BlockSpec((_BN, _D), lambda i: (i, 0)),
            pl.BlockSpec((_BN, _D), lambda i: (i, 0)),
            pl.BlockSpec((_NHEADS, _D, 32), lambda i: (0, 0, 0)),
            pl.BlockSpec((_NHEADS, _D, 32), lambda i: (0, 0, 0)),
            pl.BlockSpec((_NHEADS, 96, 32), lambda i: (0, 0, 0)),
        ],
        out_specs=pl.BlockSpec((3, _BN, _D), lambda i: (0, i, 0)),
        out_shape=jax.ShapeDtypeStruct((3, _N, _D), jnp.float32),
    )(xt, xp, pc, pp, w)


def _sc_gather(gtab, idx_chunks):
    mesh = plsc.VectorSubcoreMesh(core_axis_name="c", subcore_axis_name="s")

    @functools.partial(
        pl.kernel,
        out_type=jax.ShapeDtypeStruct((_E, _D), jnp.float32),
        mesh=mesh,
        compiler_params=pltpu.CompilerParams(use_tc_tiling_on_sc=False),
        scratch_types=[
            pltpu.VMEM((3 * _C,), jnp.int32),
            pltpu.VMEM((3 * _C,), jnp.int32),
            pltpu.VMEM((3 * _C, _D), jnp.float32),
            pltpu.VMEM((3 * _C, _D), jnp.float32),
            pltpu.VMEM((_C, _D), jnp.float32),
            pltpu.VMEM((_C, _D), jnp.float32),
            pltpu.SemaphoreType.DMA,
            pltpu.SemaphoreType.DMA,
            pltpu.SemaphoreType.DMA,
            pltpu.SemaphoreType.DMA,
            pltpu.SemaphoreType.DMA,
            pltpu.SemaphoreType.DMA,
        ],
    )
    def body(gtab_hbm, idx_hbm, out_hbm,
             idx_a, idx_b, rows_a, rows_b, obuf_a, obuf_b,
             sem_ia, sem_ib, sem_ga, sem_gb, sem_oa, sem_ob):
        wid = lax.axis_index("s") * _NC + lax.axis_index("c")
        base = pl.multiple_of(wid * _EPW, 8)
        ibase = pl.multiple_of(wid * (3 * _EPW), 8)

        def fire_idx(g, buf, sem):
            off = pl.multiple_of(ibase + g * (3 * _C), 8)
            pltpu.async_copy(idx_hbm.at[pl.ds(off, 3 * _C)], buf, sem)

        def drain_idx(buf, sem):
            pltpu.make_async_copy(
                idx_hbm.at[pl.ds(ibase, 3 * _C)], buf, sem).wait()

        def fire_gather(buf, rows, sem):
            pltpu.async_copy(gtab_hbm.at[buf], rows, sem)

        def drain_gather(buf, rows, sem):
            pltpu.make_async_copy(gtab_hbm.at[buf], rows, sem).wait()

        def compute(rows, obuf):
            def edge(e, carry):
                for j in range(_D // 16):
                    s = pl.ds(j * 16, 16)
                    x = rows[e, s] + rows[_C + e, s] + rows[2 * _C + e, s]
                    obuf[e, s] = jnp.maximum(x, x * _ALPHA)
                return carry

            lax.fori_loop(0, _C, edge, 0)

        def fire_out(g, obuf, sem):
            off = pl.multiple_of(base + g * _C, 8)
            pltpu.async_copy(obuf, out_hbm.at[pl.ds(off, _C)], sem)

        def drain_out(obuf, sem):
            pltpu.make_async_copy(obuf, out_hbm.at[pl.ds(base, _C)], sem).wait()

        fire_idx(0, idx_a, sem_ia)
        drain_idx(idx_a, sem_ia)
        fire_gather(idx_a, rows_a, sem_ga)
        fire_idx(1, idx_b, sem_ib)
        drain_idx(idx_b, sem_ib)
        fire_gather(idx_b, rows_b, sem_gb)

        def pair(p, carry):
            g = p * 2
            drain_gather(idx_a, rows_a, sem_ga)
            fire_idx(g + 2, idx_a, sem_ia)
            pl.when(p >= 1)(lambda: drain_out(obuf_a, sem_oa))
            compute(rows_a, obuf_a)
            fire_out(g, obuf_a, sem_oa)
            drain_idx(idx_a, sem_ia)
            fire_gather(idx_a, rows_a, sem_ga)
            drain_gather(idx_b, rows_b, sem_gb)
            pl.when(g + 3 < _NCHUNK)(lambda: fire_idx(g + 3, idx_b, sem_ib))
            pl.when(p >= 1)(lambda: drain_out(obuf_b, sem_ob))
            compute(rows_b, obuf_b)
            fire_out(g + 1, obuf_b, sem_ob)

            def _next_b():
                drain_idx(idx_b, sem_ib)
                fire_gather(idx_b, rows_b, sem_gb)

            pl.when(g + 3 < _NCHUNK)(_next_b)
            return carry

        lax.fori_loop(0, (_NCHUNK - 1) // 2, pair, 0)

        drain_gather(idx_a, rows_a, sem_ga)
        drain_out(obuf_a, sem_oa)
        compute(rows_a, obuf_a)
        fire_out(_NCHUNK - 1, obuf_a, sem_oa)
        drain_out(obuf_a, sem_oa)
        drain_out(obuf_b, sem_ob)

    return body(gtab, idx_chunks)


def kernel(taxPayer_feats, person_feats, item_feats, trans_adj_list,
           P_company, P_person, P_item, W_PCC):
    del item_feats, P_item
    g = _tc_proj(taxPayer_feats, person_feats, P_company, P_person, W_PCC)
    bias = jnp.array([[0], [_N], [2 * _N]], dtype=jnp.int32)
    idx = ((trans_adj_list + bias)
           .reshape(3, _NW, _NCHUNK, _C)
           .transpose(1, 2, 0, 3)
           .reshape(3 * _E))
    return _sc_gather(g.reshape(3 * _N, _D), idx)


# speedup vs baseline: 1.3877x; 1.3877x over previous
import functools

import jax
import jax.numpy as jnp
from jax import lax
from jax.experimental import pallas as pl
from jax.experimental.pallas import tpu as pltpu
from jax.experimental.pallas import tpu_sc as plsc

_N = 10000
_E = 320000
_D = 128
_NHEADS = 4
_ALPHA = 0.2

_NC = 2
_NS = 16
_NW = _NC * _NS
_EPW = _E // _NW
_C = 80
_NCHUNK = _EPW // _C

_BN = 2000


def _tc_proj(xt, xp, pc, pp, w):
    def body(xt_ref, xp_ref, pc_ref, pp_ref, w_ref, g0_ref, g1_ref, g2_ref):
        f32 = jnp.float32

        def pack_cols(y):
            bits = lax.bitcast_convert_type(
                y.astype(jnp.bfloat16).astype(f32), jnp.int32)
            return bits[:, 64:] | lax.shift_right_logical(bits[:, :64], 16)

        a0 = jnp.concatenate(
            [jnp.dot(pc_ref[h], w_ref[h, 0:32, :], preferred_element_type=f32)
             for h in range(_NHEADS)], axis=1)
        a1 = jnp.concatenate(
            [jnp.dot(pc_ref[h], w_ref[h, 32:64, :], preferred_element_type=f32)
             for h in range(_NHEADS)], axis=1)
        a2 = jnp.concatenate(
            [jnp.dot(pp_ref[h], w_ref[h, 64:96, :], preferred_element_type=f32)
             for h in range(_NHEADS)], axis=1)
        xt_blk = xt_ref[...]
        g0_ref[...] = pack_cols(jnp.dot(xt_blk, a0, preferred_element_type=f32))
        g1_ref[...] = pack_cols(jnp.dot(xt_blk, a1, preferred_element_type=f32))
        g2_ref[...] = pack_cols(jnp.dot(xp_ref[...], a2,
                                        preferred_element_type=f32))

    out = jax.ShapeDtypeStruct((_N, _D // 2), jnp.int32)
    return pl.pallas_call(
        body,
        grid=(_N // _BN,),
        in_specs=[
            pl.BlockSpec((_BN, _D), lambda i: (i, 0)),
            pl.BlockSpec((_BN, _D), lambda i: (i, 0)),
            pl.BlockSpec((_NHEADS, _D, 32), lambda i: (0, 0, 0)),
            pl.BlockSpec((_NHEADS, _D, 32), lambda i: (0, 0, 0)),
            pl.BlockSpec((_NHEADS, 96, 32), lambda i: (0, 0, 0)),
        ],
        out_specs=[
            pl.BlockSpec((_BN, _D // 2), lambda i: (i, 0)),
            pl.BlockSpec((_BN, _D // 2), lambda i: (i, 0)),
            pl.BlockSpec((_BN, _D // 2), lambda i: (i, 0)),
        ],
        out_shape=[out, out, out],
    )(xt, xp, pc, pp, w)


def _sc_gather(g0, g1, g2, idx_flat):
    mesh = plsc.VectorSubcoreMesh(core_axis_name="c", subcore_axis_name="s")

    @functools.partial(
        pl.kernel,
        out_type=jax.ShapeDtypeStruct((_E, _D), jnp.float32),
        mesh=mesh,
        compiler_params=pltpu.CompilerParams(use_tc_tiling_on_sc=False),
        scratch_types=[
            [pltpu.VMEM((_C,), jnp.int32)] * 3,
            [pltpu.VMEM((_C,), jnp.int32)] * 3,
            pltpu.VMEM((3, _C, _D // 2), jnp.int32),
            pltpu.VMEM((3, _C, _D // 2), jnp.int32),
            pltpu.VMEM((_C, _D), jnp.float32),
            pltpu.VMEM((_C, _D), jnp.float32),
            pltpu.SemaphoreType.DMA,
            pltpu.SemaphoreType.DMA,
            pltpu.SemaphoreType.DMA,
            pltpu.SemaphoreType.DMA,
            pltpu.SemaphoreType.DMA,
            pltpu.SemaphoreType.DMA,
        ],
    )
    def body(g0_hbm, g1_hbm, g2_hbm, idx_hbm, out_hbm,
             idx_a, idx_b, rows_a, rows_b, obuf_a, obuf_b,
             sem_ia, sem_ib, sem_ga, sem_gb, sem_oa, sem_ob):
        wid = lax.axis_index("s") * _NC + lax.axis_index("c")
        base = pl.multiple_of(wid * _EPW, 8)
        tables = (g0_hbm, g1_hbm, g2_hbm)

        def fire_idx(g, bufs, sem):
            for k in range(3):
                off = pl.multiple_of(k * _E + base + g * _C, 8)
                pltpu.async_copy(idx_hbm.at[pl.ds(off, _C)], bufs[k], sem)

        def drain_idx(bufs, sem):
            for k in range(3):
                pltpu.make_async_copy(
                    idx_hbm.at[pl.ds(base, _C)], bufs[k], sem).wait()

        def fire_gathers(bufs, rows, sem):
            for k in range(3):
                pltpu.async_copy(tables[k].at[bufs[k]], rows.at[k], sem)

        def drain_gathers(bufs, rows, sem):
            for k in range(3):
                pltpu.make_async_copy(tables[k].at[bufs[k]], rows.at[k], sem
                                      ).wait()

        def compute(rows, obuf):
            mask = jnp.int32(-65536)

            @plsc.parallel_loop(0, _C, unroll=4)
            def edge(e):
                for j in range(_D // 32):
                    s = pl.ds(j * 16, 16)
                    ws = [rows[k, e, s] for k in range(3)]
                    lo = hi = None
                    for w in ws:
                        l = lax.bitcast_convert_type(
                            lax.shift_left(w, 16), jnp.float32)
                        h = lax.bitcast_convert_type(w & mask, jnp.float32)
                        lo = l if lo is None else lo + l
                        hi = h if hi is None else hi + h
                    obuf[e, pl.ds(j * 16, 16)] = jnp.maximum(lo, lo * _ALPHA)
                    obuf[e, pl.ds(64 + j * 16, 16)] = jnp.maximum(hi, hi * _ALPHA)

        def fire_out(g, obuf, sem):
            off = pl.multiple_of(base + g * _C, 8)
            pltpu.async_copy(obuf, out_hbm.at[pl.ds(off, _C)], sem)

        def drain_out(obuf, sem):
            pltpu.make_async_copy(obuf, out_hbm.at[pl.ds(base, _C)], sem).wait()

        fire_idx(0, idx_a, sem_ia)
        drain_idx(idx_a, sem_ia)
        fire_gathers(idx_a, rows_a, sem_ga)
        fire_idx(1, idx_b, sem_ib)
        drain_idx(idx_b, sem_ib)
        fire_gathers(idx_b, rows_b, sem_gb)

        def pair(p, carry):
            g = p * 2
            drain_gathers(idx_a, rows_a, sem_ga)
            fire_idx(g + 2, idx_a, sem_ia)
            pl.when(p >= 1)(lambda: drain_out(obuf_a, sem_oa))
            compute(rows_a, obuf_a)
            fire_out(g, obuf_a, sem_oa)
            drain_idx(idx_a, sem_ia)
            fire_gathers(idx_a, rows_a, sem_ga)
            drain_gathers(idx_b, rows_b, sem_gb)
            pl.when(g + 3 < _NCHUNK)(lambda: fire_idx(g + 3, idx_b, sem_ib))
            pl.when(p >= 1)(lambda: drain_out(obuf_b, sem_ob))
            compute(rows_b, obuf_b)
            fire_out(g + 1, obuf_b, sem_ob)

            def _next_b():
                drain_idx(idx_b, sem_ib)
                fire_gathers(idx_b, rows_b, sem_gb)

            pl.when(g + 3 < _NCHUNK)(_next_b)
            return carry

        lax.fori_loop(0, (_NCHUNK - 1) // 2, pair, 0)

        drain_gathers(idx_a, rows_a, sem_ga)
        drain_out(obuf_a, sem_oa)
        compute(rows_a, obuf_a)
        fire_out(_NCHUNK - 1, obuf_a, sem_oa)
        drain_out(obuf_a, sem_oa)
        drain_out(obuf_b, sem_ob)

    return body(g0, g1, g2, idx_flat)


def kernel(taxPayer_feats, person_feats, item_feats, trans_adj_list,
           P_company, P_person, P_item, W_PCC):
    del item_feats, P_item
    g0, g1, g2 = _tc_proj(taxPayer_feats, person_feats, P_company, P_person, W_PCC)
    return _sc_gather(g0, g1, g2, trans_adj_list.reshape(3 * _E))


# speedup vs baseline: 1.3921x; 1.0032x over previous
import functools

import jax
import jax.numpy as jnp
from jax import lax
from jax.experimental import pallas as pl
from jax.experimental.pallas import tpu as pltpu
from jax.experimental.pallas import tpu_sc as plsc

_N = 10000
_E = 320000
_D = 128
_NHEADS = 4
_ALPHA = 0.2

_NC = 2
_NS = 16
_NW = _NC * _NS
_EPW = _E // _NW
_C = 80
_NCHUNK = _EPW // _C

_BN = 2000


def _tc_proj(xt, xp, pc, pp, w):
    def body(xt_ref, xp_ref, pc_ref, pp_ref, w_ref, g0_ref, g1_ref, g2_ref):
        f32 = jnp.float32

        def pack_cols(y):
            bits = lax.bitcast_convert_type(
                y.astype(jnp.bfloat16).astype(f32), jnp.int32)
            return bits[:, 64:] | lax.shift_right_logical(bits[:, :64], 16)

        a0 = jnp.concatenate(
            [jnp.dot(pc_ref[h], w_ref[h, 0:32, :], preferred_element_type=f32)
             for h in range(_NHEADS)], axis=1)
        a1 = jnp.concatenate(
            [jnp.dot(pc_ref[h], w_ref[h, 32:64, :], preferred_element_type=f32)
             for h in range(_NHEADS)], axis=1)
        a2 = jnp.concatenate(
            [jnp.dot(pp_ref[h], w_ref[h, 64:96, :], preferred_element_type=f32)
             for h in range(_NHEADS)], axis=1)
        xt_blk = xt_ref[...]
        g0_ref[...] = pack_cols(jnp.dot(xt_blk, a0, preferred_element_type=f32))
        g1_ref[...] = pack_cols(jnp.dot(xt_blk, a1, preferred_element_type=f32))
        g2_ref[...] = pack_cols(jnp.dot(xp_ref[...], a2,
                                        preferred_element_type=f32))

    out = jax.ShapeDtypeStruct((_N, _D // 2), jnp.int32)
    return pl.pallas_call(
        body,
        grid=(_N // _BN,),
        in_specs=[
            pl.BlockSpec((_BN, _D), lambda i: (i, 0)),
            pl.BlockSpec((_BN, _D), lambda i: (i, 0)),
            pl.BlockSpec((_NHEADS, _D, 32), lambda i: (0, 0, 0)),
            pl.BlockSpec((_NHEADS, _D, 32), lambda i: (0, 0, 0)),
            pl.BlockSpec((_NHEADS, 96, 32), lambda i: (0, 0, 0)),
        ],
        out_specs=[
            pl.BlockSpec((_BN, _D // 2), lambda i: (i, 0)),
            pl.BlockSpec((_BN, _D // 2), lambda i: (i, 0)),
            pl.BlockSpec((_BN, _D // 2), lambda i: (i, 0)),
        ],
        out_shape=[out, out, out],
    )(xt, xp, pc, pp, w)


def _sc_gather(g0, g1, g2, idx_flat):
    mesh = plsc.VectorSubcoreMesh(core_axis_name="c", subcore_axis_name="s")

    @functools.partial(
        pl.kernel,
        out_type=jax.ShapeDtypeStruct((_E, _D), jnp.float32),
        mesh=mesh,
        compiler_params=pltpu.CompilerParams(use_tc_tiling_on_sc=False),
        scratch_types=[
            [pltpu.VMEM((_C,), jnp.int32)] * 3,
            [pltpu.VMEM((_C,), jnp.int32)] * 3,
            pltpu.VMEM((3, _C, _D // 2), jnp.int32),
            pltpu.VMEM((3, _C, _D // 2), jnp.int32),
            pltpu.VMEM((_C, _D), jnp.float32),
            pltpu.VMEM((_C, _D), jnp.float32),
            pltpu.SemaphoreType.DMA,
            pltpu.SemaphoreType.DMA,
            pltpu.SemaphoreType.DMA,
            pltpu.SemaphoreType.DMA,
            pltpu.SemaphoreType.DMA,
            pltpu.SemaphoreType.DMA,
        ],
    )
    def body(g0_hbm, g1_hbm, g2_hbm, idx_hbm, out_hbm,
             idx_a, idx_b, rows_a, rows_b, obuf_a, obuf_b,
             sem_ia, sem_ib, sem_ga, sem_gb, sem_oa, sem_ob):
        wid = lax.axis_index("s") * _NC + lax.axis_index("c")
        base = pl.multiple_of(wid * _EPW, 8)
        tables = (g0_hbm, g1_hbm, g2_hbm)

        def fire_idx(g, bufs, sem):
            for k in range(3):
                off = pl.multiple_of(k * _E + base + g * _C, 8)
                pltpu.async_copy(idx_hbm.at[pl.ds(off, _C)], bufs[k], sem)

        def drain_idx(bufs, sem):
            for k in range(3):
                pltpu.make_async_copy(
                    idx_hbm.at[pl.ds(base, _C)], bufs[k], sem).wait()

        def fire_gathers(bufs, rows, sem):
            for k in range(3):
                pltpu.async_copy(tables[k].at[bufs[k]], rows.at[k], sem)

        def drain_gathers(bufs, rows, sem):
            for k in range(3):
                pltpu.make_async_copy(tables[k].at[bufs[k]], rows.at[k], sem
                                      ).wait()

        def compute(rows, obuf):
            mask = jnp.int32(-65536)

            @plsc.parallel_loop(0, _C, unroll=8)
            def edge(e):
                for j in range(_D // 32):
                    s = pl.ds(j * 16, 16)
                    ws = [rows[k, e, s] for k in range(3)]
                    lo = hi = None
                    for w in ws:
                        l = lax.bitcast_convert_type(
                            lax.shift_left(w, 16), jnp.float32)
                        h = lax.bitcast_convert_type(w & mask, jnp.float32)
                        lo = l if lo is None else lo + l
                        hi = h if hi is None else hi + h
                    obuf[e, pl.ds(j * 16, 16)] = jnp.maximum(lo, lo * _ALPHA)
                    obuf[e, pl.ds(64 + j * 16, 16)] = jnp.maximum(hi, hi * _ALPHA)

        def fire_out(g, obuf, sem):
            off = pl.multiple_of(base + g * _C, 8)
            pltpu.async_copy(obuf, out_hbm.at[pl.ds(off, _C)], sem)

        def drain_out(obuf, sem):
            pltpu.make_async_copy(obuf, out_hbm.at[pl.ds(base, _C)], sem).wait()

        fire_idx(0, idx_a, sem_ia)
        drain_idx(idx_a, sem_ia)
        fire_gathers(idx_a, rows_a, sem_ga)
        fire_idx(1, idx_b, sem_ib)
        drain_idx(idx_b, sem_ib)
        fire_gathers(idx_b, rows_b, sem_gb)

        def pair(p, carry):
            g = p * 2
            drain_gathers(idx_a, rows_a, sem_ga)
            fire_idx(g + 2, idx_a, sem_ia)
            pl.when(p >= 1)(lambda: drain_out(obuf_a, sem_oa))
            compute(rows_a, obuf_a)
            fire_out(g, obuf_a, sem_oa)
            drain_idx(idx_a, sem_ia)
            fire_gathers(idx_a, rows_a, sem_ga)
            drain_gathers(idx_b, rows_b, sem_gb)
            pl.when(g + 3 < _NCHUNK)(lambda: fire_idx(g + 3, idx_b, sem_ib))
            pl.when(p >= 1)(lambda: drain_out(obuf_b, sem_ob))
            compute(rows_b, obuf_b)
            fire_out(g + 1, obuf_b, sem_ob)

            def _next_b():
                drain_idx(idx_b, sem_ib)
                fire_gathers(idx_b, rows_b, sem_gb)

            pl.when(g + 3 < _NCHUNK)(_next_b)
            return carry

        lax.fori_loop(0, (_NCHUNK - 1) // 2, pair, 0)

        drain_gathers(idx_a, rows_a, sem_ga)
        drain_out(obuf_a, sem_oa)
        compute(rows_a, obuf_a)
        fire_out(_NCHUNK - 1, obuf_a, sem_oa)
        drain_out(obuf_a, sem_oa)
        drain_out(obuf_b, sem_ob)

    return body(g0, g1, g2, idx_flat)


def kernel(taxPayer_feats, person_feats, item_feats, trans_adj_list,
           P_company, P_person, P_item, W_PCC):
    del item_feats, P_item
    g0, g1, g2 = _tc_proj(taxPayer_feats, person_feats, P_company, P_person, W_PCC)
    return _sc_gather(g0, g1, g2, trans_adj_list.reshape(3 * _E))
